# baseline (speedup 1.0000x reference)
import functools

import jax
import jax.numpy as jnp
from jax import lax
from jax.experimental import pallas as pl
from jax.experimental.pallas import tpu as pltpu
from jax.experimental.pallas import tpu_sc as plsc

_BATCH = 16384
_EMBED_DIM = 64
_NUM_CORES = 2
_NUM_SUBCORES = 16
_NW = _NUM_CORES * _NUM_SUBCORES
_B_PER_W = _BATCH // _NW
_CHUNK = 128
_NCH = _B_PER_W // _CHUNK


@functools.partial(
    pl.kernel,
    out_type=jax.ShapeDtypeStruct((_BATCH, _EMBED_DIM), jnp.float32),
    mesh=plsc.VectorSubcoreMesh(core_axis_name="c", subcore_axis_name="s"),
    scratch_types=[
        pltpu.VMEM((_NCH, _CHUNK), jnp.int32),
        pltpu.VMEM((_B_PER_W, _EMBED_DIM), jnp.float32),
        pltpu.SemaphoreType.DMA,
    ],
    compiler_params=pltpu.CompilerParams(use_tc_tiling_on_sc=False),
)
def _embedding_gather(idx_hbm, table_hbm, out_hbm, idx_v, rows_v, sem):
    wid = lax.axis_index("s") * _NUM_CORES + lax.axis_index("c")
    pltpu.sync_copy(idx_hbm.at[pl.ds(wid * _NCH, _NCH)], idx_v)
    copies = []
    for j in range(_NCH):
        copies.append(
            pltpu.async_copy(
                table_hbm.at[idx_v.at[j]],
                rows_v.at[pl.ds(j * _CHUNK, _CHUNK)],
                sem,
            )
        )
    for c in copies:
        c.wait()
    pltpu.sync_copy(rows_v, out_hbm.at[pl.ds(wid * _B_PER_W, _B_PER_W)])


def kernel(user_id, table):
    idx = user_id.astype(jnp.int32).reshape(_BATCH // _CHUNK, _CHUNK)
    return _embedding_gather(idx, table)


# speedup vs baseline: 2.5266x; 2.5266x over previous
import functools

import jax
import jax.numpy as jnp
from jax import lax
from jax.experimental import pallas as pl
from jax.experimental.pallas import tpu as pltpu
from jax.experimental.pallas import tpu_sc as plsc

_BATCH = 16384
_EMBED_DIM = 64
_NUM_CORES = 2
_NUM_SUBCORES = 16
_NW = _NUM_CORES * _NUM_SUBCORES
_B_PER_W = _BATCH // _NW
_LANES = 16
_DEPTH = 4


@functools.partial(
    pl.kernel,
    out_type=jax.ShapeDtypeStruct((_BATCH, _EMBED_DIM), jnp.float32),
    mesh=plsc.VectorSubcoreMesh(core_axis_name="c", subcore_axis_name="s"),
    scratch_types=[
        pltpu.VMEM((_B_PER_W // 128, 128), jnp.int32),
        [pltpu.VMEM((_EMBED_DIM, 128), jnp.float32) for _ in range(_DEPTH)],
        pltpu.VMEM((_B_PER_W, _EMBED_DIM), jnp.float32),
        [pltpu.SemaphoreType.DMA for _ in range(_DEPTH)],
    ],
    compiler_params=pltpu.CompilerParams(
        disable_bounds_checks=True, needs_layout_passes=False
    ),
)
def _embedding_gather(idx_hbm, table_t_hbm, out_hbm, idx_s, panels, rows_v, sems):
    wid = lax.axis_index("s") * _NUM_CORES + lax.axis_index("c")
    base = wid * _B_PER_W
    pltpu.sync_copy(idx_hbm.at[pl.ds(wid * (_B_PER_W // 128), _B_PER_W // 128)], idx_s)

    def uid(u):
        j = u >> 7
        g = (u >> 4) & 7
        m = u & 15
        v = idx_s[j, pl.ds(g * _LANES, _LANES)]
        sel = jnp.where(lax.iota(jnp.int32, _LANES) == m, v, jnp.int32(0))
        return jnp.sum(sel)

    def fire(u, b):
        col = pl.multiple_of((uid(u) >> 7) << 7, 128)
        pltpu.async_copy(table_t_hbm.at[:, pl.ds(col, 128)], panels[b], sems[b])

    def drain(b):
        pltpu.make_async_copy(
            table_t_hbm.at[:, pl.ds(0, 128)], panels[b], sems[b]
        ).wait()

    def extract(u, b):
        lane = jnp.full((_LANES,), uid(u) & 127, dtype=jnp.int32)
        for k in range(_EMBED_DIM // _LANES):
            dims = lax.iota(jnp.int32, _LANES) + (k * _LANES)
            vals = plsc.load_gather(panels[b], [dims, lane])
            rows_v[u, pl.ds(k * _LANES, _LANES)] = vals

    for b in range(_DEPTH):
        fire(b, b)

    def step(i):
        for b in range(_DEPTH):
            u = i * _DEPTH + b
            drain(b)
            extract(u, b)

            @pl.when(u + _DEPTH < _B_PER_W)
            def _():
                fire(u + _DEPTH, b)

    pl.loop(0, _B_PER_W // _DEPTH)(step)
    pltpu.sync_copy(rows_v, out_hbm.at[pl.ds(base, _B_PER_W), :])


def kernel(user_id, table):
    idx = user_id.astype(jnp.int32).reshape(_BATCH // 128, 128)
    return _embedding_gather(idx, table.T)


# speedup vs baseline: 3.0333x; 1.2005x over previous
import functools

import jax
import jax.numpy as jnp
from jax import lax
from jax.experimental import pallas as pl
from jax.experimental.pallas import tpu as pltpu
from jax.experimental.pallas import tpu_sc as plsc

_BATCH = 16384
_EMBED_DIM = 64
_NUM_CORES = 2
_NUM_SUBCORES = 16
_NW = _NUM_CORES * _NUM_SUBCORES
_B_PER_W = _BATCH // _NW
_LANES = 16
_DEPTH = 8


@functools.partial(
    pl.kernel,
    out_type=jax.ShapeDtypeStruct((_EMBED_DIM, _BATCH), jnp.float32),
    mesh=plsc.VectorSubcoreMesh(core_axis_name="c", subcore_axis_name="s"),
    scratch_types=[
        pltpu.VMEM((_B_PER_W // 128, 128), jnp.int32),
        [pltpu.VMEM((_EMBED_DIM, 128), jnp.float32) for _ in range(_DEPTH)],
        pltpu.VMEM((_DEPTH, _LANES), jnp.int32),
        pltpu.VMEM((_EMBED_DIM, _B_PER_W), jnp.float32),
        [pltpu.SemaphoreType.DMA for _ in range(_DEPTH)],
    ],
    compiler_params=pltpu.CompilerParams(
        disable_bounds_checks=True, needs_layout_passes=False
    ),
)
def _embedding_gather(
    idx_hbm, table_t_hbm, out_hbm, idx_s, panels, lane_ring, cols_v, sems
):
    wid = lax.axis_index("s") * _NUM_CORES + lax.axis_index("c")
    pltpu.sync_copy(idx_hbm.at[pl.ds(wid * (_B_PER_W // 128), _B_PER_W // 128)], idx_s)

    def fire(u, b):
        v = idx_s[u >> 7, pl.ds(((u >> 4) & 7) * _LANES, _LANES)]
        sel = jnp.where(lax.iota(jnp.int32, _LANES) == (u & 15), v, jnp.int32(0))
        r = jnp.sum(sel)
        lane_ring[b, :] = jnp.full((_LANES,), r & 127, dtype=jnp.int32)
        col = pl.multiple_of((r >> 7) << 7, 128)
        pltpu.async_copy(table_t_hbm.at[:, pl.ds(col, 128)], panels[b], sems[b])

    def drain(b):
        pltpu.make_async_copy(
            table_t_hbm.at[:, pl.ds(0, 128)], panels[b], sems[b]
        ).wait()

    def extract(u, b):
        lane = lane_ring[b, :]
        upos = jnp.full((_LANES,), u, dtype=jnp.int32)
        for k in range(_EMBED_DIM // _LANES):
            dims = lax.iota(jnp.int32, _LANES) + (k * _LANES)
            vals = plsc.load_gather(panels[b], [dims, lane])
            plsc.store_scatter(cols_v, [dims, upos], vals)

    for b in range(_DEPTH):
        fire(b, b)

    def step(i):
        for b in range(_DEPTH):
            u = i * _DEPTH + b
            drain(b)
            extract(u, b)

            @pl.when(u + _DEPTH < _B_PER_W)
            def _():
                fire(u + _DEPTH, b)

    pl.loop(0, _B_PER_W // _DEPTH)(step)
    pltpu.sync_copy(cols_v, out_hbm.at[:, pl.ds(wid * _B_PER_W, _B_PER_W)])


def kernel(user_id, table):
    idx = user_id.astype(jnp.int32).reshape(_BATCH // 128, 128)
    return _embedding_gather(idx, table.T).T


# speedup vs baseline: 3.0547x; 1.0071x over previous
import functools

import jax
import jax.numpy as jnp
from jax import lax
from jax.experimental import pallas as pl
from jax.experimental.pallas import tpu as pltpu
from jax.experimental.pallas import tpu_sc as plsc

_BATCH = 16384
_EMBED_DIM = 64
_NUM_CORES = 2
_NUM_SUBCORES = 16
_NW = _NUM_CORES * _NUM_SUBCORES
_B_PER_W = _BATCH // _NW
_LANES = 16
_DEPTH = 8


@functools.partial(
    pl.kernel,
    out_type=jax.ShapeDtypeStruct((_EMBED_DIM, _BATCH), jnp.float32),
    mesh=plsc.VectorSubcoreMesh(core_axis_name="c", subcore_axis_name="s"),
    scratch_types=[
        pltpu.VMEM((_B_PER_W // 128, 128), jnp.int32),
        [pltpu.VMEM((_EMBED_DIM, 128), jnp.float32) for _ in range(_DEPTH)],
        pltpu.VMEM((_DEPTH, _LANES), jnp.int32),
        pltpu.VMEM((_EMBED_DIM, _B_PER_W), jnp.float32),
        [pltpu.SemaphoreType.DMA for _ in range(_DEPTH)],
    ],
    compiler_params=pltpu.CompilerParams(
        disable_bounds_checks=True, needs_layout_passes=False
    ),
)
def _embedding_gather(
    idx_hbm, table_t_hbm, out_hbm, idx_s, panels, lane_ring, cols_v, sems
):
    wid = lax.axis_index("s") * _NUM_CORES + lax.axis_index("c")
    pltpu.sync_copy(idx_hbm.at[pl.ds(wid * (_B_PER_W // 128), _B_PER_W // 128)], idx_s)

    def fire(u, b):
        v = idx_s[u >> 7, pl.ds(((u >> 4) & 7) * _LANES, _LANES)]
        sel = jnp.where(lax.iota(jnp.int32, _LANES) == (u & 15), v, jnp.int32(0))
        r = jnp.sum(sel)
        lane_ring[b, :] = jnp.full((_LANES,), r & 127, dtype=jnp.int32)
        col = pl.multiple_of((r >> 7) << 7, 128)
        pltpu.async_copy(
            table_t_hbm.at[pl.ds(0, 32), pl.ds(col, 128)],
            panels[b].at[pl.ds(0, 32)],
            sems[b],
        )
        pltpu.async_copy(
            table_t_hbm.at[pl.ds(32, 32), pl.ds(col, 128)],
            panels[b].at[pl.ds(32, 32)],
            sems[b],
        )

    def drain(b):
        pltpu.make_async_copy(
            table_t_hbm.at[:, pl.ds(0, 128)], panels[b], sems[b]
        ).wait()

    def extract(u, b):
        lane = lane_ring[b, :]
        upos = jnp.full((_LANES,), u, dtype=jnp.int32)
        for k in range(_EMBED_DIM // _LANES):
            dims = lax.iota(jnp.int32, _LANES) + (k * _LANES)
            vals = plsc.load_gather(panels[b], [dims, lane])
            plsc.store_scatter(cols_v, [dims, upos], vals)

    for b in range(_DEPTH):
        fire(b, b)

    def step(i):
        for b in range(_DEPTH):
            u = i * _DEPTH + b
            drain(b)
            extract(u, b)

            @pl.when(u + _DEPTH < _B_PER_W)
            def _():
                fire(u + _DEPTH, b)

    pl.loop(0, _B_PER_W // _DEPTH)(step)
    pltpu.sync_copy(cols_v, out_hbm.at[:, pl.ds(wid * _B_PER_W, _B_PER_W)])


def kernel(user_id, table):
    idx = user_id.astype(jnp.int32).reshape(_BATCH // 128, 128)
    return _embedding_gather(idx, table.T).T
